# baseline (speedup 1.0000x reference)
import functools

import jax
import jax.numpy as jnp
from jax import lax
from jax.experimental import pallas as pl
from jax.experimental.pallas import tpu as pltpu
from jax.experimental.pallas import tpu_sc as plsc

_INV_STEPS = 1.0 / 16.0
_NBUF = 8
_DEPTH = _NBUF - 2


@functools.cache
def _sc_kernel(steps, rows, cols):
    info = plsc.get_sparse_core_info()
    nc, ns, lanes = info.num_cores, info.num_subcores, info.num_lanes
    nw = nc * ns
    rw = rows // nw
    ch = rw // _NBUF
    nch = rw // ch
    nt = steps * nch
    ngroups = nt // _NBUF
    groups = ch * cols // lanes
    mesh = plsc.VectorSubcoreMesh(core_axis_name="c", subcore_axis_name="s")

    @functools.partial(
        pl.kernel,
        out_type=jax.ShapeDtypeStruct((steps, rows, cols), jnp.float32),
        mesh=mesh,
        scratch_types=[
            pltpu.VMEM((lanes,), jnp.int32),
            [pltpu.VMEM((ch, cols), jnp.float32) for _ in range(_NBUF)],
            pltpu.VMEM((ch, cols), jnp.float32),
            [pltpu.SemaphoreType.DMA for _ in range(_NBUF)],
            [pltpu.SemaphoreType.DMA for _ in range(_NBUF)],
        ],
    )
    def k(iv_hbm, d_hbm, nd_hbm, o_hbm, iv_v, bufs, ndbuf, in_sems, out_sems):
        wid = lax.axis_index("s") * nc + lax.axis_index("c")
        base = wid * rw
        pltpu.sync_copy(iv_hbm, iv_v)
        it = iv_v[...][0]

        def chunk_coords(t):
            return (t // nch + wid) % steps, base + (t % nch) * ch

        def start_in(t, b):
            s, lo = chunk_coords(t)
            pltpu.make_async_copy(
                d_hbm.at[s, pl.ds(lo, ch)], bufs[b], in_sems[b]
            ).start()

        def wait_in(b):
            pltpu.make_async_copy(
                d_hbm.at[0, pl.ds(base, ch)], bufs[b], in_sems[b]
            ).wait()

        def start_out(t, b):
            s, lo = chunk_coords(t)
            pltpu.make_async_copy(
                bufs[b], o_hbm.at[s, pl.ds(lo, ch)], out_sems[b]
            ).start()

        def wait_out(b):
            pltpu.make_async_copy(
                bufs[b], o_hbm.at[0, pl.ds(base, ch)], out_sems[b]
            ).wait()

        def process(t, b):
            wait_in(b)
            s, lo = chunk_coords(t)

            @pl.when(s == it)
            def _():
                pltpu.sync_copy(nd_hbm.at[pl.ds(lo, ch)], ndbuf)

                def add_body(u, acc):
                    r = u // (cols // lanes)
                    jc = (u % (cols // lanes)) * lanes
                    bufs[b][r, pl.ds(jc, lanes)] = (
                        bufs[b][r, pl.ds(jc, lanes)]
                        + ndbuf[r, pl.ds(jc, lanes)] * _INV_STEPS
                    )
                    return acc

                lax.fori_loop(0, groups, add_body, 0)

            start_out(t, b)

        for b in range(_DEPTH):
            start_in(b, b)
        for b in range(_NBUF):
            if b >= 2:
                wait_out((b + _DEPTH) % _NBUF)
            start_in(b + _DEPTH, (b + _DEPTH) % _NBUF)
            process(b, b)

        def group_body(g, carry):
            for b in range(_NBUF):
                t = g * _NBUF + b
                wait_out((b + _DEPTH) % _NBUF)
                start_in(t + _DEPTH, (b + _DEPTH) % _NBUF)
                process(t, b)
            return carry

        lax.fori_loop(1, ngroups - 1, group_body, 0)

        for b in range(_NBUF):
            t = (ngroups - 1) * _NBUF + b
            if t + _DEPTH < nt:
                wait_out((b + _DEPTH) % _NBUF)
                start_in(t + _DEPTH, (b + _DEPTH) % _NBUF)
            process(t, b)
        for b in range(_NBUF):
            wait_out(b)

    return k


def kernel(data, new_data, i):
    steps, rows, cols = data.shape
    iv = jnp.full((16,), jnp.asarray(i, jnp.int32))
    return _sc_kernel(steps, rows, cols)(iv, data, new_data)


# speedup vs baseline: 1.0179x; 1.0179x over previous
import functools

import jax
import jax.numpy as jnp
from jax import lax
from jax.experimental import pallas as pl
from jax.experimental.pallas import tpu as pltpu
from jax.experimental.pallas import tpu_sc as plsc

_INV_STEPS = 1.0 / 16.0
_NBUF = 4


@functools.cache
def _sc_kernel(steps, rows, cols):
    info = plsc.get_sparse_core_info()
    nc, ns, lanes = info.num_cores, info.num_subcores, info.num_lanes
    nw = nc * ns
    rw = rows // nw
    ch = rw // _NBUF
    nch = rw // ch
    nt = steps * nch
    ngroups = nt // _NBUF
    groups = ch * cols // lanes
    mesh = plsc.VectorSubcoreMesh(core_axis_name="c", subcore_axis_name="s")

    @functools.partial(
        pl.kernel,
        out_type=jax.ShapeDtypeStruct((steps, rows, cols), jnp.float32),
        mesh=mesh,
        scratch_types=[
            pltpu.VMEM((lanes,), jnp.int32),
            [pltpu.VMEM((ch, cols), jnp.float32) for _ in range(_NBUF)],
            pltpu.VMEM((ch, cols), jnp.float32),
            [pltpu.SemaphoreType.DMA for _ in range(_NBUF)],
            [pltpu.SemaphoreType.DMA for _ in range(_NBUF)],
        ],
    )
    def k(iv_hbm, d_hbm, nd_hbm, o_hbm, iv_v, bufs, ndbuf, in_sems, out_sems):
        wid = lax.axis_index("s") * nc + lax.axis_index("c")
        base = wid * rw
        pltpu.sync_copy(iv_hbm, iv_v)
        it = iv_v[...][0]

        def chunk_coords(t):
            return (t // nch + wid) % steps, base + (t % nch) * ch

        def start_in(t, b):
            s, lo = chunk_coords(t)
            pltpu.make_async_copy(
                d_hbm.at[s, pl.ds(lo, ch)], bufs[b], in_sems[b]
            ).start()

        def wait_in(b):
            pltpu.make_async_copy(
                d_hbm.at[0, pl.ds(base, ch)], bufs[b], in_sems[b]
            ).wait()

        def start_out(t, b):
            s, lo = chunk_coords(t)
            pltpu.make_async_copy(
                bufs[b], o_hbm.at[s, pl.ds(lo, ch)], out_sems[b]
            ).start()

        def wait_out(b):
            pltpu.make_async_copy(
                bufs[b], o_hbm.at[0, pl.ds(base, ch)], out_sems[b]
            ).wait()

        def process(t, b):
            wait_in(b)
            s, lo = chunk_coords(t)

            @pl.when(s == it)
            def _():
                pltpu.sync_copy(nd_hbm.at[pl.ds(lo, ch)], ndbuf)

                def add_body(u, acc):
                    r = u // (cols // lanes)
                    jc = (u % (cols // lanes)) * lanes
                    bufs[b][r, pl.ds(jc, lanes)] = (
                        bufs[b][r, pl.ds(jc, lanes)]
                        + ndbuf[r, pl.ds(jc, lanes)] * _INV_STEPS
                    )
                    return acc

                lax.fori_loop(0, groups, add_body, 0)

            start_out(t, b)

        start_in(0, 0)
        start_in(1, 1)
        for b in range(_NBUF):
            if b >= 2:
                wait_out((b + 2) % _NBUF)
            start_in(b + 2, (b + 2) % _NBUF)
            process(b, b)

        def group_body(g, carry):
            for b in range(_NBUF):
                t = g * _NBUF + b
                wait_out((b + 2) % _NBUF)
                start_in(t + 2, (b + 2) % _NBUF)
                process(t, b)
            return carry

        lax.fori_loop(1, ngroups - 1, group_body, 0)

        for b in range(_NBUF):
            t = (ngroups - 1) * _NBUF + b
            if t + 2 < nt:
                wait_out((b + 2) % _NBUF)
                start_in(t + 2, (b + 2) % _NBUF)
            process(t, b)
        for b in range(_NBUF):
            wait_out(b)

    return k


def kernel(data, new_data, i):
    steps, rows, cols = data.shape
    iv = jnp.full((16,), jnp.asarray(i, jnp.int32))
    return _sc_kernel(steps, rows, cols)(iv, data, new_data)
